# baseline (speedup 1.0000x reference)
import functools

import jax
import jax.numpy as jnp
from jax import lax
from jax.experimental import pallas as pl
from jax.experimental.pallas import tpu as pltpu
from jax.experimental.pallas import tpu_sc as plsc

_B = 16
_N = 10000
_D = 128
_E = 320000
_SENTINEL = 1 << 30

_NC = 2
_NS = 16
_NW = _NC * _NS
_CW = 312
_TB = _NW * _CW
_TAIL = _N - _TB
_EC = _E // _NW


def _dotT(a, b):
    return lax.dot_general(a, b, (((0,), (0,)), ((), ())),
                           preferred_element_type=jnp.float32)


def _prep_body(x_ref, batch_ref, p_ref, src_ref, nb_ref, kvec_ref,
               xsc_ref):
    x = x_ref[:]
    bat = batch_ref[:]
    p = p_ref[:]

    score = jnp.dot(x, p, preferred_element_type=jnp.float32)
    g_row = lax.broadcasted_iota(jnp.int32, (1, _B), 1).astype(jnp.float32)
    batf = bat.astype(jnp.float32)
    mask = batf == g_row
    maskf = mask.astype(jnp.float32)

    neg = jnp.float32(-3.0e38)
    smax = jnp.max(jnp.where(mask, score, neg), axis=0, keepdims=True)
    smax = jnp.where(smax > neg * 0.5, smax, 0.0)
    smax_col = smax.reshape(_B, 1)
    e = jnp.exp(score - jnp.dot(maskf, smax_col,
                                preferred_element_type=jnp.float32))
    ones_col = jnp.ones((_N, 1), dtype=jnp.float32)
    seg = _dotT(maskf, jnp.concatenate([e, ones_col], axis=1))
    ssum_col = seg[:, 0:1]
    n_col = seg[:, 1:2]
    denom = jnp.dot(maskf, ssum_col, preferred_element_type=jnp.float32)
    xsc_ref[:] = x * (e / denom)

    tri = (lax.broadcasted_iota(jnp.int32, (_B, _B), 1) <
           lax.broadcasted_iota(jnp.int32, (_B, _B), 0)).astype(jnp.float32)
    cum_col = jnp.dot(tri, n_col, preferred_element_type=jnp.float32)
    kk_col = jnp.floor((n_col + 1.0) * 0.5)
    kcum_col = jnp.dot(tri, kk_col, preferred_element_type=jnp.float32)
    dcum_col = cum_col - kcum_col
    K_tot = jnp.sum(kk_col, axis=0, keepdims=True)

    jcol = lax.broadcasted_iota(jnp.int32, (_N, 1), 0).astype(jnp.float32)
    ones16 = jnp.ones((_B, 1), dtype=jnp.float32)
    kcum_row = kcum_col.reshape(1, _B)
    dcum_row = dcum_col.reshape(1, _B)

    gk = jnp.dot((jcol >= kcum_row).astype(jnp.float32), ones16,
                 preferred_element_type=jnp.float32) - 1.0
    ohkf = (gk == g_row).astype(jnp.float32)
    src_keep = jnp.dot(ohkf, cum_col - kcum_col,
                       preferred_element_type=jnp.float32) + jcol

    jj = jcol - K_tot
    gd = jnp.dot((jj >= dcum_row).astype(jnp.float32), ones16,
                 preferred_element_type=jnp.float32) - 1.0
    ohdf = (gd == g_row).astype(jnp.float32)
    src_drop = jnp.dot(ohdf, cum_col + kk_col - dcum_col,
                       preferred_element_type=jnp.float32) + jj

    keep = jcol < K_tot
    src_ref[:] = jnp.where(keep, src_keep, src_drop).astype(jnp.int32)
    nb_ref[:] = jnp.where(keep, gk, jnp.float32(_B)).astype(jnp.int32)

    maxb = jnp.max(batf, axis=0, keepdims=True)
    kval = jnp.where(maxb == jnp.float32(_B - 1), K_tot,
                     jnp.float32(_SENTINEL)).astype(jnp.int32)
    kvec_ref[:] = jnp.broadcast_to(kval, (1, _B))


_prep = pl.pallas_call(
    _prep_body,
    out_shape=(
        jax.ShapeDtypeStruct((_N, 1), jnp.int32),
        jax.ShapeDtypeStruct((_N, 1), jnp.int32),
        jax.ShapeDtypeStruct((1, _B), jnp.int32),
        jax.ShapeDtypeStruct((_N, _D), jnp.float32),
    ),
)


@functools.cache
def _get_sc_kernel():
    @functools.partial(
        pl.kernel,
        mesh=plsc.VectorSubcoreMesh(core_axis_name="c", subcore_axis_name="s"),
        out_type=(
            jax.ShapeDtypeStruct((_N, _D), jnp.float32),
            jax.ShapeDtypeStruct((_E,), jnp.int32),
        ),
        scratch_types=[
            pltpu.VMEM((_CW,), jnp.int32),
            pltpu.VMEM((_CW, _D), jnp.float32),
            pltpu.VMEM((_EC,), jnp.int32),
            pltpu.VMEM((_EC,), jnp.int32),
            pltpu.VMEM((_B,), jnp.int32),
            pltpu.SemaphoreType.DMA,
            pltpu.SemaphoreType.DMA,
            pltpu.SemaphoreType.DMA,
        ],
    )
    def _sc_gather_edges(xsc_hbm, src_hbm, s_hbm, r_hbm, kvec_hbm,
                         xs_out, s2_out, idx_v, rows_v, sv, rv, kv,
                         esem, gsem, osem):
        wid = lax.axis_index("s") * _NC + lax.axis_index("c")
        ebase = wid * _EC
        nbase = wid * _CW

        e1 = pltpu.async_copy(s_hbm.at[pl.ds(ebase, _EC)], sv, esem)
        e2 = pltpu.async_copy(r_hbm.at[pl.ds(ebase, _EC)], rv, esem)
        pltpu.sync_copy(kvec_hbm, kv)

        pltpu.sync_copy(src_hbm.at[pl.ds(nbase, _CW)], idx_v)
        gathers = []
        for off in (0, 104, 208):
            gathers.append(pltpu.async_copy(
                xsc_hbm.at[idx_v.at[pl.ds(off, 104)]],
                rows_v.at[pl.ds(off, 104)], gsem))

        e1.wait()
        e2.wait()
        kvv = kv[...]

        def ebody(i, carry):
            sl = pl.ds(i * 16, 16)
            s = sv[sl]
            r = rv[sl]
            m = (s >= kvv) | (r >= kvv)
            sv[sl] = jnp.where(m, r, s)
            return carry

        lax.fori_loop(0, _EC // 16, ebody, 0)
        eo = pltpu.async_copy(sv, s2_out.at[pl.ds(ebase, _EC)], osem)

        for g in gathers:
            g.wait()
        pltpu.sync_copy(rows_v, xs_out.at[pl.ds(nbase, _CW)])

        @pl.when(wid == 0)
        def _():
            pltpu.sync_copy(src_hbm.at[pl.ds(_TB, _TAIL)],
                            idx_v.at[pl.ds(0, _TAIL)])
            pltpu.async_copy(xsc_hbm.at[idx_v.at[pl.ds(0, _TAIL)]],
                             rows_v.at[pl.ds(0, _TAIL)], gsem).wait()
            pltpu.sync_copy(rows_v.at[pl.ds(0, _TAIL)],
                            xs_out.at[pl.ds(_TB, _TAIL)])

        eo.wait()

    return _sc_gather_edges


def kernel(x, senders, receivers, batch, p):
    batch_col = batch.reshape(_N, 1)
    p_col = p.reshape(_D, 1)

    src_col, nb_col, kvec, xsc = _prep(x, batch_col, p_col)

    src_flat = src_col.reshape(_N)
    xs, senders2 = _get_sc_kernel()(xsc, src_flat, senders, receivers,
                                    kvec.reshape(_B))
    new_batch = nb_col.reshape(_N)
    return (xs, senders2, receivers, new_batch)
